# Initial kernel scaffold; baseline (speedup 1.0000x reference)
import jax
import jax.numpy as jnp
from jax.experimental import pallas as pl


def kernel(user_emb, item_emb, edge_index):
    raise NotImplementedError("write your pallas kernel here")


# speedup vs baseline: 13.0494x; 13.0494x over previous
import jax
import jax.numpy as jnp
from jax import lax
from jax.experimental import pallas as pl
from jax.experimental.pallas import tpu as pltpu
from jax.experimental.pallas import tpu_sc as plsc

N_USERS = 25000
N_ITEMS = 25000
NN = N_USERS + N_ITEMS
D = 64
DH = 32
E = 800000
N_LAYERS = 3

NC, NS, L = 2, 16, 16
EPT = E // NS
CH = 400
NCHUNK = EPT // CH
NNP = 50176
NPT = NNP // NS
DCH = 112
NDCH = NPT // DCH


def _zero_rows(ref, nrows, ncols):
    z = jnp.zeros((L,), jnp.float32)

    def body(i, _):
        for j in range(ncols // L):
            ref[i, pl.ds(j * L, L)] = z
        return 0

    lax.fori_loop(0, nrows, body, 0)


def _fill_ones(ref, nrows, ncols):
    o = jnp.ones((L,), jnp.float32)

    def body(i, _):
        for j in range(ncols // L):
            ref[i, pl.ds(j * L, L)] = o
        return 0

    lax.fori_loop(0, nrows, body, 0)


def _sc_body(x0_hbm, row_hbm, col_hbm, out_hbm,
             y_hbm, dinv_hbm, acc,
             rowbuf, colbuf, rows, abuf, bbuf, dvbuf, zbuf, sem):
    cid = lax.axis_index("c")
    tid = lax.axis_index("s")
    loc0 = tid * NPT
    gbase = cid * NNP + loc0
    yoff = jnp.zeros((L,), jnp.int32) + cid * NNP

    _zero_rows(zbuf, DCH, DH)
    _fill_ones(rows, CH, DH)
    for c in range(NDCH):
        pltpu.sync_copy(zbuf, acc.at[pl.ds(loc0 + c * DCH, DCH)])
    plsc.subcore_barrier()

    def deg_chunk(ch, _):
        off = tid * EPT + ch * CH
        pltpu.sync_copy(row_hbm.at[pl.ds(off, CH)], rowbuf)
        pltpu.sync_copy(rows, acc.at[rowbuf], add=True)
        return 0

    lax.fori_loop(0, NCHUNK, deg_chunk, 0)
    plsc.subcore_barrier()

    magic = jnp.zeros((L,), jnp.int32) + 0x5F3759DF

    def init_chunk(c, _):
        g = gbase + c * DCH
        gl = loc0 + c * DCH
        pltpu.sync_copy(acc.at[pl.ds(gl, DCH)], dvbuf)
        pltpu.sync_copy(x0_hbm.at[pl.ds(g, DCH)], abuf)

        def rowstep(i, _):
            for j in range(DH // L):
                x = dvbuf[i, pl.ds(j * L, L)]
                iv = plsc.bitcast(x, jnp.int32)
                y = plsc.bitcast(magic - (iv >> 1), jnp.float32)
                for _ in range(3):
                    y = y * (1.5 - 0.5 * x * y * y)
                dv = jnp.where(x > 0.5, y, 0.0)
                dvbuf[i, pl.ds(j * L, L)] = dv
                vx = abuf[i, pl.ds(j * L, L)]
                bbuf[i, pl.ds(j * L, L)] = 0.25 * vx
                abuf[i, pl.ds(j * L, L)] = dv * vx
            return 0

        lax.fori_loop(0, DCH, rowstep, 0)
        pltpu.sync_copy(dvbuf, dinv_hbm.at[pl.ds(g, DCH)])
        pltpu.sync_copy(bbuf, out_hbm.at[pl.ds(g, DCH)])
        pltpu.sync_copy(abuf, y_hbm.at[pl.ds(g, DCH)])
        pltpu.sync_copy(zbuf, acc.at[pl.ds(gl, DCH)])
        return 0

    lax.fori_loop(0, NDCH, init_chunk, 0)
    plsc.subcore_barrier()

    for layer in range(N_LAYERS):
        def edge_chunk(ch, _):
            off = tid * EPT + ch * CH
            pltpu.sync_copy(col_hbm.at[pl.ds(off, CH)], colbuf)
            pltpu.sync_copy(row_hbm.at[pl.ds(off, CH)], rowbuf)

            def adj(j, _):
                colbuf[pl.ds(j * L, L)] = colbuf[pl.ds(j * L, L)] + yoff
                return 0

            lax.fori_loop(0, CH // L, adj, 0)
            pltpu.async_copy(y_hbm.at[colbuf], rows, sem).wait()
            pltpu.sync_copy(rows, acc.at[rowbuf], add=True)
            return 0

        lax.fori_loop(0, NCHUNK, edge_chunk, 0)
        plsc.subcore_barrier()

        last = layer == N_LAYERS - 1

        def dense_chunk(c, _):
            g = gbase + c * DCH
            gl = loc0 + c * DCH
            pltpu.sync_copy(acc.at[pl.ds(gl, DCH)], abuf)
            pltpu.sync_copy(out_hbm.at[pl.ds(g, DCH)], bbuf)
            pltpu.sync_copy(dinv_hbm.at[pl.ds(g, DCH)], dvbuf)

            def rowstep(i, _):
                for j in range(DH // L):
                    a = abuf[i, pl.ds(j * L, L)]
                    dv = dvbuf[i, pl.ds(j * L, L)]
                    xn = dv * a
                    bbuf[i, pl.ds(j * L, L)] = (
                        bbuf[i, pl.ds(j * L, L)] + 0.25 * xn)
                    if not last:
                        abuf[i, pl.ds(j * L, L)] = dv * xn
                return 0

            lax.fori_loop(0, DCH, rowstep, 0)
            pltpu.sync_copy(bbuf, out_hbm.at[pl.ds(g, DCH)])
            if not last:
                pltpu.sync_copy(abuf, y_hbm.at[pl.ds(g, DCH)])
                pltpu.sync_copy(zbuf, acc.at[pl.ds(gl, DCH)])
            return 0

        lax.fori_loop(0, NDCH, dense_chunk, 0)
        if not last:
            plsc.subcore_barrier()


@jax.jit
def _lightgcn_sc(x0_cat, row, col):
    mesh = plsc.VectorSubcoreMesh(
        core_axis_name="c", subcore_axis_name="s",
        num_cores=NC, num_subcores=NS)
    return pl.kernel(
        _sc_body,
        out_type=jax.ShapeDtypeStruct((NC * NNP, DH), jnp.float32),
        mesh=mesh,
        compiler_params=pltpu.CompilerParams(
            needs_layout_passes=False, use_tc_tiling_on_sc=False),
        scratch_types=[
            pltpu.HBM((NC * NNP, DH), jnp.float32),
            pltpu.HBM((NC * NNP, DH), jnp.float32),
            pltpu.VMEM_SHARED((NNP, DH), jnp.float32),
            pltpu.VMEM((CH,), jnp.int32),
            pltpu.VMEM((CH,), jnp.int32),
            pltpu.VMEM((CH, DH), jnp.float32),
            pltpu.VMEM((DCH, DH), jnp.float32),
            pltpu.VMEM((DCH, DH), jnp.float32),
            pltpu.VMEM((DCH, DH), jnp.float32),
            pltpu.VMEM((DCH, DH), jnp.float32),
            pltpu.SemaphoreType.DMA,
        ],
    )(x0_cat, row, col)


def kernel(user_emb, item_emb, edge_index):
    all0 = jnp.concatenate([user_emb, item_emb], axis=0)
    pad = jnp.zeros((NNP - NN, DH), jnp.float32)
    x0_cat = jnp.concatenate([all0[:, :DH], pad, all0[:, DH:], pad], axis=0)
    out = _lightgcn_sc(x0_cat, edge_index[0], edge_index[1])
    fin = jnp.concatenate([out[:NN], out[NNP:NNP + NN]], axis=1)
    return fin[:N_USERS], fin[N_USERS:]


# speedup vs baseline: 15.3683x; 1.1777x over previous
import jax
import jax.numpy as jnp
from jax import lax
from jax.experimental import pallas as pl
from jax.experimental.pallas import tpu as pltpu
from jax.experimental.pallas import tpu_sc as plsc

N_USERS = 25000
N_ITEMS = 25000
NN = N_USERS + N_ITEMS
D = 64
DH = 32
E = 800000
N_LAYERS = 3

NC, NS, L = 2, 16, 16
EPT = E // NS
CH = 400
NCHUNK = EPT // CH
NPAIR = NCHUNK // 2
NNP = 50176
NPT = NNP // NS
DCH = 28
NDCH = NPT // DCH


def _zero_rows(ref, nrows, ncols):
    z = jnp.zeros((L,), jnp.float32)

    def body(i, _):
        for j in range(ncols // L):
            ref[i, pl.ds(j * L, L)] = z
        return 0

    lax.fori_loop(0, nrows, body, 0)


def _fill_ones(ref, nrows, ncols):
    o = jnp.ones((L,), jnp.float32)

    def body(i, _):
        for j in range(ncols // L):
            ref[i, pl.ds(j * L, L)] = o
        return 0

    lax.fori_loop(0, nrows, body, 0)


def _sc_body(x0_hbm, row_hbm, col_hbm, out_hbm,
             y_hbm, dinv_hbm, acc,
             rowbuf0, rowbuf1, colbuf0, colbuf1, rows0, rows1,
             abuf, bbuf, dvbuf,
             sema, semb, semc, semg0, semg1, sems0, sems1):
    cid = lax.axis_index("c")
    tid = lax.axis_index("s")
    loc0 = tid * NPT
    gbase = cid * NNP + loc0
    yoff = jnp.zeros((L,), jnp.int32) + cid * NNP

    def _adj(cb):
        def body(j, _):
            cb[pl.ds(j * L, L)] = cb[pl.ds(j * L, L)] + yoff
            return 0

        lax.fori_loop(0, CH // L, body, 0)

    _zero_rows(bbuf, DCH, DH)
    _fill_ones(rows0, CH, DH)

    def zfire(c, _):
        pltpu.async_copy(bbuf, acc.at[pl.ds(loc0 + c * DCH, DCH)], sema)
        return 0

    def zdrain(c, _):
        pltpu.make_async_copy(
            bbuf, acc.at[pl.ds(loc0 + c * DCH, DCH)], sema).wait()
        return 0

    lax.fori_loop(0, NDCH, zfire, 0)
    lax.fori_loop(0, NDCH, zdrain, 0)
    plsc.subcore_barrier()

    def hist_pair(p, _):
        off0 = tid * EPT + (2 * p) * CH
        pltpu.sync_copy(row_hbm.at[pl.ds(off0, CH)], rowbuf0)
        s0 = pltpu.async_copy(rows0, acc.at[rowbuf0], sems0, add=True)
        pltpu.sync_copy(row_hbm.at[pl.ds(off0 + CH, CH)], rowbuf1)
        s1 = pltpu.async_copy(rows0, acc.at[rowbuf1], sems1, add=True)
        s0.wait()
        s1.wait()
        return 0

    lax.fori_loop(0, NPAIR, hist_pair, 0)
    offt = tid * EPT + (NCHUNK - 1) * CH
    pltpu.sync_copy(row_hbm.at[pl.ds(offt, CH)], rowbuf0)
    pltpu.sync_copy(rows0, acc.at[rowbuf0], add=True)
    plsc.subcore_barrier()

    magic = jnp.zeros((L,), jnp.int32) + 0x5F3759DF

    def init_chunk(c, _):
        g = gbase + c * DCH
        gl = loc0 + c * DCH
        ra = pltpu.async_copy(acc.at[pl.ds(gl, DCH)], dvbuf, sema)
        rb = pltpu.async_copy(x0_hbm.at[pl.ds(g, DCH)], abuf, semb)
        ra.wait()
        rb.wait()

        def rowstep(i, _):
            for j in range(DH // L):
                x = dvbuf[i, pl.ds(j * L, L)]
                iv = plsc.bitcast(x, jnp.int32)
                y = plsc.bitcast(magic - (iv >> 1), jnp.float32)
                for _ in range(3):
                    y = y * (1.5 - 0.5 * x * y * y)
                dv = jnp.where(x > 0.5, y, 0.0)
                dvbuf[i, pl.ds(j * L, L)] = dv
                vx = abuf[i, pl.ds(j * L, L)]
                bbuf[i, pl.ds(j * L, L)] = 0.25 * vx
                abuf[i, pl.ds(j * L, L)] = dv * vx
            return 0

        lax.fori_loop(0, DCH, rowstep, 0)
        wa = pltpu.async_copy(dvbuf, dinv_hbm.at[pl.ds(g, DCH)], sema)
        wb = pltpu.async_copy(bbuf, out_hbm.at[pl.ds(g, DCH)], semb)
        wc = pltpu.async_copy(abuf, y_hbm.at[pl.ds(g, DCH)], semc)
        wa.wait()
        wb.wait()
        wc.wait()
        _zero_rows(bbuf, DCH, DH)
        pltpu.sync_copy(bbuf, acc.at[pl.ds(gl, DCH)])
        return 0

    lax.fori_loop(0, NDCH, init_chunk, 0)
    plsc.subcore_barrier()

    for layer in range(N_LAYERS):
        def edge_pair(p, _):
            off0 = tid * EPT + (2 * p) * CH
            pltpu.sync_copy(col_hbm.at[pl.ds(off0, CH)], colbuf0)
            _adj(colbuf0)
            g0 = pltpu.async_copy(y_hbm.at[colbuf0], rows0, semg0)
            pltpu.sync_copy(row_hbm.at[pl.ds(off0, CH)], rowbuf0)
            pltpu.sync_copy(col_hbm.at[pl.ds(off0 + CH, CH)], colbuf1)
            _adj(colbuf1)
            g1 = pltpu.async_copy(y_hbm.at[colbuf1], rows1, semg1)
            pltpu.sync_copy(row_hbm.at[pl.ds(off0 + CH, CH)], rowbuf1)
            g0.wait()
            s0 = pltpu.async_copy(rows0, acc.at[rowbuf0], sems0, add=True)
            g1.wait()
            s1 = pltpu.async_copy(rows1, acc.at[rowbuf1], sems1, add=True)
            s0.wait()
            s1.wait()
            return 0

        lax.fori_loop(0, NPAIR, edge_pair, 0)
        offt2 = tid * EPT + (NCHUNK - 1) * CH
        pltpu.sync_copy(col_hbm.at[pl.ds(offt2, CH)], colbuf0)
        _adj(colbuf0)
        pltpu.async_copy(y_hbm.at[colbuf0], rows0, semg0).wait()
        pltpu.sync_copy(row_hbm.at[pl.ds(offt2, CH)], rowbuf0)
        pltpu.sync_copy(rows0, acc.at[rowbuf0], add=True)
        plsc.subcore_barrier()

        last = layer == N_LAYERS - 1

        def dense_chunk(c, _):
            g = gbase + c * DCH
            gl = loc0 + c * DCH
            ra = pltpu.async_copy(acc.at[pl.ds(gl, DCH)], abuf, sema)
            rb = pltpu.async_copy(out_hbm.at[pl.ds(g, DCH)], bbuf, semb)
            rc = pltpu.async_copy(dinv_hbm.at[pl.ds(g, DCH)], dvbuf, semc)
            ra.wait()
            rb.wait()
            rc.wait()

            def rowstep(i, _):
                for j in range(DH // L):
                    a = abuf[i, pl.ds(j * L, L)]
                    dv = dvbuf[i, pl.ds(j * L, L)]
                    xn = dv * a
                    bbuf[i, pl.ds(j * L, L)] = (
                        bbuf[i, pl.ds(j * L, L)] + 0.25 * xn)
                    if not last:
                        abuf[i, pl.ds(j * L, L)] = dv * xn
                return 0

            lax.fori_loop(0, DCH, rowstep, 0)
            wb = pltpu.async_copy(bbuf, out_hbm.at[pl.ds(g, DCH)], semb)
            if not last:
                wa = pltpu.async_copy(abuf, y_hbm.at[pl.ds(g, DCH)], sema)
                wa.wait()
            wb.wait()
            if not last:
                _zero_rows(bbuf, DCH, DH)
                pltpu.sync_copy(bbuf, acc.at[pl.ds(gl, DCH)])
            return 0

        lax.fori_loop(0, NDCH, dense_chunk, 0)
        if not last:
            plsc.subcore_barrier()


@jax.jit
def _lightgcn_sc(x0_cat, row, col):
    mesh = plsc.VectorSubcoreMesh(
        core_axis_name="c", subcore_axis_name="s",
        num_cores=NC, num_subcores=NS)
    return pl.kernel(
        _sc_body,
        out_type=jax.ShapeDtypeStruct((NC * NNP, DH), jnp.float32),
        mesh=mesh,
        compiler_params=pltpu.CompilerParams(
            needs_layout_passes=False, use_tc_tiling_on_sc=False),
        scratch_types=[
            pltpu.HBM((NC * NNP, DH), jnp.float32),
            pltpu.HBM((NC * NNP, DH), jnp.float32),
            pltpu.VMEM_SHARED((NNP, DH), jnp.float32),
            pltpu.VMEM((CH,), jnp.int32),
            pltpu.VMEM((CH,), jnp.int32),
            pltpu.VMEM((CH,), jnp.int32),
            pltpu.VMEM((CH,), jnp.int32),
            pltpu.VMEM((CH, DH), jnp.float32),
            pltpu.VMEM((CH, DH), jnp.float32),
            pltpu.VMEM((DCH, DH), jnp.float32),
            pltpu.VMEM((DCH, DH), jnp.float32),
            pltpu.VMEM((DCH, DH), jnp.float32),
            pltpu.SemaphoreType.DMA,
            pltpu.SemaphoreType.DMA,
            pltpu.SemaphoreType.DMA,
            pltpu.SemaphoreType.DMA,
            pltpu.SemaphoreType.DMA,
            pltpu.SemaphoreType.DMA,
            pltpu.SemaphoreType.DMA,
        ],
    )(x0_cat, row, col)


def kernel(user_emb, item_emb, edge_index):
    all0 = jnp.concatenate([user_emb, item_emb], axis=0)
    pad = jnp.zeros((NNP - NN, DH), jnp.float32)
    x0_cat = jnp.concatenate([all0[:, :DH], pad, all0[:, DH:], pad], axis=0)
    out = _lightgcn_sc(x0_cat, edge_index[0], edge_index[1])
    fin = jnp.concatenate([out[:NN], out[NNP:NNP + NN]], axis=1)
    return fin[:N_USERS], fin[N_USERS:]


# speedup vs baseline: 16.9503x; 1.1029x over previous
import jax
import jax.numpy as jnp
from jax import lax
from jax.experimental import pallas as pl
from jax.experimental.pallas import tpu as pltpu
from jax.experimental.pallas import tpu_sc as plsc

N_USERS = 25000
N_ITEMS = 25000
NN = N_USERS + N_ITEMS
D = 64
DH = 32
E = 800000
N_LAYERS = 3

NC, NS, L = 2, 16, 16
EPT = E // NS
CH = 400
NCHUNK = EPT // CH
NPAIR = NCHUNK // 2
NNP = 50176
NPT = NNP // NS
DCH = 64
NDCH = NPT // DCH


def _zero_rows(ref, nrows, ncols):
    z = jnp.zeros((L,), jnp.float32)

    def body(i, _):
        for j in range(ncols // L):
            ref[i, pl.ds(j * L, L)] = z
        return 0

    lax.fori_loop(0, nrows, body, 0)


def _fill_ones(ref, nrows, ncols):
    o = jnp.ones((L,), jnp.float32)

    def body(i, _):
        for j in range(ncols // L):
            ref[i, pl.ds(j * L, L)] = o
        return 0

    lax.fori_loop(0, nrows, body, 0)


def _sc_body(x0_hbm, row_hbm, col_hbm, out_hbm,
             y_hbm, dinv_hbm, acc,
             rowbuf0, rowbuf1, colbuf0, colbuf1, rows0, rows1, bbuf,
             sema, semb, semc, semg0, semg1, sems0, sems1):
    cid = lax.axis_index("c")
    tid = lax.axis_index("s")
    loc0 = tid * NPT
    gbase = cid * NNP + loc0
    yoff = jnp.zeros((L,), jnp.int32) + cid * NNP

    def _adj(cb):
        def body(j, _):
            cb[pl.ds(j * L, L)] = cb[pl.ds(j * L, L)] + yoff
            return 0

        lax.fori_loop(0, CH // L, body, 0)

    _zero_rows(bbuf, DCH, DH)
    _fill_ones(rows0, CH, DH)

    def zfire(c, _):
        pltpu.async_copy(bbuf, acc.at[pl.ds(loc0 + c * DCH, DCH)], sema)
        return 0

    def zdrain(c, _):
        pltpu.make_async_copy(
            bbuf, acc.at[pl.ds(loc0 + c * DCH, DCH)], sema).wait()
        return 0

    lax.fori_loop(0, NDCH, zfire, 0)
    lax.fori_loop(0, NDCH, zdrain, 0)
    plsc.subcore_barrier()

    def hist_pair(p, _):
        off0 = tid * EPT + (2 * p) * CH
        pltpu.sync_copy(row_hbm.at[pl.ds(off0, CH)], rowbuf0)
        s0 = pltpu.async_copy(rows0, acc.at[rowbuf0], sems0, add=True)
        pltpu.sync_copy(row_hbm.at[pl.ds(off0 + CH, CH)], rowbuf1)
        s1 = pltpu.async_copy(rows0, acc.at[rowbuf1], sems1, add=True)
        s0.wait()
        s1.wait()
        return 0

    lax.fori_loop(0, NPAIR, hist_pair, 0)
    offt = tid * EPT + (NCHUNK - 1) * CH
    pltpu.sync_copy(row_hbm.at[pl.ds(offt, CH)], rowbuf0)
    pltpu.sync_copy(rows0, acc.at[rowbuf0], add=True)
    plsc.subcore_barrier()

    magic = jnp.zeros((L,), jnp.int32) + 0x5F3759DF

    def init_chunk(c, _):
        g = gbase + c * DCH
        gl = loc0 + c * DCH
        ra = pltpu.async_copy(acc.at[pl.ds(gl, DCH)], rows1.at[pl.ds(0, DCH)],
                              sema)
        rb = pltpu.async_copy(x0_hbm.at[pl.ds(g, DCH)], rows0.at[pl.ds(0, DCH)],
                              semb)
        ra.wait()
        rb.wait()

        def rowstep(i, _):
            for j in range(DH // L):
                x = rows1[i, pl.ds(j * L, L)]
                iv = plsc.bitcast(x, jnp.int32)
                y = plsc.bitcast(magic - (iv >> 1), jnp.float32)
                for _ in range(3):
                    y = y * (1.5 - 0.5 * x * y * y)
                dv = jnp.where(x > 0.5, y, 0.0)
                rows1[i, pl.ds(j * L, L)] = dv
                vx = rows0[i, pl.ds(j * L, L)]
                bbuf[i, pl.ds(j * L, L)] = 0.25 * vx
                rows0[i, pl.ds(j * L, L)] = dv * vx
            return 0

        lax.fori_loop(0, DCH, rowstep, 0)
        wa = pltpu.async_copy(rows1.at[pl.ds(0, DCH)],
                              dinv_hbm.at[pl.ds(g, DCH)], sema)
        wb = pltpu.async_copy(bbuf, out_hbm.at[pl.ds(g, DCH)], semb)
        wc = pltpu.async_copy(rows0.at[pl.ds(0, DCH)],
                              y_hbm.at[pl.ds(g, DCH)], semc)
        wa.wait()
        wb.wait()
        wc.wait()
        _zero_rows(bbuf, DCH, DH)
        pltpu.sync_copy(bbuf, acc.at[pl.ds(gl, DCH)])
        return 0

    lax.fori_loop(0, NDCH, init_chunk, 0)
    plsc.subcore_barrier()

    for layer in range(N_LAYERS):
        def edge_pair(p, _):
            off0 = tid * EPT + (2 * p) * CH
            pltpu.sync_copy(col_hbm.at[pl.ds(off0, CH)], colbuf0)
            _adj(colbuf0)
            g0 = pltpu.async_copy(y_hbm.at[colbuf0], rows0, semg0)
            pltpu.sync_copy(row_hbm.at[pl.ds(off0, CH)], rowbuf0)
            pltpu.sync_copy(col_hbm.at[pl.ds(off0 + CH, CH)], colbuf1)
            _adj(colbuf1)
            g1 = pltpu.async_copy(y_hbm.at[colbuf1], rows1, semg1)
            pltpu.sync_copy(row_hbm.at[pl.ds(off0 + CH, CH)], rowbuf1)
            g0.wait()
            s0 = pltpu.async_copy(rows0, acc.at[rowbuf0], sems0, add=True)
            g1.wait()
            s1 = pltpu.async_copy(rows1, acc.at[rowbuf1], sems1, add=True)
            s0.wait()
            s1.wait()
            return 0

        lax.fori_loop(0, NPAIR, edge_pair, 0)
        offt2 = tid * EPT + (NCHUNK - 1) * CH
        pltpu.sync_copy(col_hbm.at[pl.ds(offt2, CH)], colbuf0)
        _adj(colbuf0)
        pltpu.async_copy(y_hbm.at[colbuf0], rows0, semg0).wait()
        pltpu.sync_copy(row_hbm.at[pl.ds(offt2, CH)], rowbuf0)
        pltpu.sync_copy(rows0, acc.at[rowbuf0], add=True)
        plsc.subcore_barrier()

        last = layer == N_LAYERS - 1

        def dense_chunk(c, _):
            g = gbase + c * DCH
            gl = loc0 + c * DCH
            ra = pltpu.async_copy(acc.at[pl.ds(gl, DCH)],
                                  rows0.at[pl.ds(0, DCH)], sema)
            rb = pltpu.async_copy(out_hbm.at[pl.ds(g, DCH)], bbuf, semb)
            rc = pltpu.async_copy(dinv_hbm.at[pl.ds(g, DCH)],
                                  rows1.at[pl.ds(0, DCH)], semc)
            ra.wait()
            rb.wait()
            rc.wait()

            def rowstep(i, _):
                for j in range(DH // L):
                    a = rows0[i, pl.ds(j * L, L)]
                    dv = rows1[i, pl.ds(j * L, L)]
                    xn = dv * a
                    bbuf[i, pl.ds(j * L, L)] = (
                        bbuf[i, pl.ds(j * L, L)] + 0.25 * xn)
                    if not last:
                        rows0[i, pl.ds(j * L, L)] = dv * xn
                return 0

            lax.fori_loop(0, DCH, rowstep, 0)
            wb = pltpu.async_copy(bbuf, out_hbm.at[pl.ds(g, DCH)], semb)
            if not last:
                wa = pltpu.async_copy(rows0.at[pl.ds(0, DCH)],
                                      y_hbm.at[pl.ds(g, DCH)], sema)
                wa.wait()
            wb.wait()
            if not last:
                _zero_rows(bbuf, DCH, DH)
                pltpu.sync_copy(bbuf, acc.at[pl.ds(gl, DCH)])
            return 0

        lax.fori_loop(0, NDCH, dense_chunk, 0)
        if not last:
            plsc.subcore_barrier()


@jax.jit
def _lightgcn_sc(x0_cat, row, col):
    mesh = plsc.VectorSubcoreMesh(
        core_axis_name="c", subcore_axis_name="s",
        num_cores=NC, num_subcores=NS)
    return pl.kernel(
        _sc_body,
        out_type=jax.ShapeDtypeStruct((NC * NNP, DH), jnp.float32),
        mesh=mesh,
        compiler_params=pltpu.CompilerParams(
            needs_layout_passes=False, use_tc_tiling_on_sc=False),
        scratch_types=[
            pltpu.HBM((NC * NNP, DH), jnp.float32),
            pltpu.HBM((NC * NNP, DH), jnp.float32),
            pltpu.VMEM_SHARED((NNP, DH), jnp.float32),
            pltpu.VMEM((CH,), jnp.int32),
            pltpu.VMEM((CH,), jnp.int32),
            pltpu.VMEM((CH,), jnp.int32),
            pltpu.VMEM((CH,), jnp.int32),
            pltpu.VMEM((CH, DH), jnp.float32),
            pltpu.VMEM((CH, DH), jnp.float32),
            pltpu.VMEM((DCH, DH), jnp.float32),
            pltpu.SemaphoreType.DMA,
            pltpu.SemaphoreType.DMA,
            pltpu.SemaphoreType.DMA,
            pltpu.SemaphoreType.DMA,
            pltpu.SemaphoreType.DMA,
            pltpu.SemaphoreType.DMA,
            pltpu.SemaphoreType.DMA,
        ],
    )(x0_cat, row, col)


def kernel(user_emb, item_emb, edge_index):
    all0 = jnp.concatenate([user_emb, item_emb], axis=0)
    pad = jnp.zeros((NNP - NN, DH), jnp.float32)
    x0_cat = jnp.concatenate([all0[:, :DH], pad, all0[:, DH:], pad], axis=0)
    out = _lightgcn_sc(x0_cat, edge_index[0], edge_index[1])
    fin = jnp.concatenate([out[:NN], out[NNP:NNP + NN]], axis=1)
    return fin[:N_USERS], fin[N_USERS:]


# speedup vs baseline: 17.6918x; 1.0437x over previous
import jax
import jax.numpy as jnp
from jax import lax
from jax.experimental import pallas as pl
from jax.experimental.pallas import tpu as pltpu
from jax.experimental.pallas import tpu_sc as plsc

N_USERS = 25000
N_ITEMS = 25000
NN = N_USERS + N_ITEMS
D = 64
DH = 32
E = 800000
N_LAYERS = 3

NC, NS, L = 2, 16, 16
EPT = E // NS
CH = 400
NCHUNK = EPT // CH
NPAIR = NCHUNK // 2
NNP = 50176
NPT = NNP // NS
DCH = 64
NDCH = NPT // DCH


def _zero_rows(ref, nrows, ncols):
    z = jnp.zeros((L,), jnp.float32)

    def body(i, _):
        for j in range(ncols // L):
            ref[i, pl.ds(j * L, L)] = z
        return 0

    lax.fori_loop(0, nrows, body, 0)


def _fill_ones(ref, nrows, ncols):
    o = jnp.ones((L,), jnp.float32)

    def body(i, _):
        for j in range(ncols // L):
            ref[i, pl.ds(j * L, L)] = o
        return 0

    lax.fori_loop(0, nrows, body, 0)


def _sc_body(x0_hbm, row_hbm, col_hbm, out_hbm,
             y_hbm, dinv_hbm, acc,
             rowbuf0, rowbuf1, colbuf0, colbuf1, rows0, rows1, bbuf,
             sema, semb, semc, semg0, semg1, sems0, sems1):
    cid = lax.axis_index("c")
    tid = lax.axis_index("s")
    loc0 = tid * NPT
    gbase = cid * NNP + loc0
    yoff = jnp.zeros((L,), jnp.int32) + cid * NNP

    def _adj(cb):
        def body(j, _):
            cb[pl.ds(j * L, L)] = cb[pl.ds(j * L, L)] + yoff
            return 0

        lax.fori_loop(0, CH // L, body, 0)

    _zero_rows(bbuf, DCH, DH)
    _fill_ones(rows0, CH, DH)

    def zfire(c, _):
        pltpu.async_copy(bbuf, acc.at[pl.ds(loc0 + c * DCH, DCH)], sema)
        return 0

    def zdrain(c, _):
        pltpu.make_async_copy(
            bbuf, acc.at[pl.ds(loc0 + c * DCH, DCH)], sema).wait()
        return 0

    lax.fori_loop(0, NDCH, zfire, 0)
    lax.fori_loop(0, NDCH, zdrain, 0)
    plsc.subcore_barrier()

    def hist_pair(p, _):
        off0 = tid * EPT + (2 * p) * CH
        pltpu.sync_copy(row_hbm.at[pl.ds(off0, CH)], rowbuf0)
        s0 = pltpu.async_copy(rows0, acc.at[rowbuf0], sems0, add=True)
        pltpu.sync_copy(row_hbm.at[pl.ds(off0 + CH, CH)], rowbuf1)
        s1 = pltpu.async_copy(rows0, acc.at[rowbuf1], sems1, add=True)
        s0.wait()
        s1.wait()
        return 0

    lax.fori_loop(0, NPAIR, hist_pair, 0)
    offt = tid * EPT + (NCHUNK - 1) * CH
    pltpu.sync_copy(row_hbm.at[pl.ds(offt, CH)], rowbuf0)
    pltpu.sync_copy(rows0, acc.at[rowbuf0], add=True)
    plsc.subcore_barrier()

    magic = jnp.zeros((L,), jnp.int32) + 0x5F3759DF

    def init_chunk(c, _):
        g = gbase + c * DCH
        gl = loc0 + c * DCH
        ra = pltpu.async_copy(acc.at[pl.ds(gl, DCH)], rows1.at[pl.ds(0, DCH)],
                              sema)
        rb = pltpu.async_copy(x0_hbm.at[pl.ds(g, DCH)], rows0.at[pl.ds(0, DCH)],
                              semb)
        ra.wait()
        rb.wait()

        def rowstep(i, _):
            for j in range(DH // L):
                x = rows1[i, pl.ds(j * L, L)]
                iv = plsc.bitcast(x, jnp.int32)
                y = plsc.bitcast(magic - (iv >> 1), jnp.float32)
                for _ in range(3):
                    y = y * (1.5 - 0.5 * x * y * y)
                dv = jnp.where(x > 0.5, y, 0.0)
                rows1[i, pl.ds(j * L, L)] = dv
                vx = rows0[i, pl.ds(j * L, L)]
                bbuf[i, pl.ds(j * L, L)] = 0.25 * vx
                rows0[i, pl.ds(j * L, L)] = dv * vx
            return 0

        lax.fori_loop(0, DCH, rowstep, 0)
        wa = pltpu.async_copy(rows1.at[pl.ds(0, DCH)],
                              dinv_hbm.at[pl.ds(g, DCH)], sema)
        wb = pltpu.async_copy(
            bbuf, out_hbm.at[pl.ds(gl, DCH), pl.ds(cid * DH, DH)], semb)
        wc = pltpu.async_copy(rows0.at[pl.ds(0, DCH)],
                              y_hbm.at[pl.ds(g, DCH)], semc)
        wa.wait()
        wb.wait()
        wc.wait()
        _zero_rows(bbuf, DCH, DH)
        pltpu.sync_copy(bbuf, acc.at[pl.ds(gl, DCH)])
        return 0

    lax.fori_loop(0, NDCH, init_chunk, 0)
    plsc.subcore_barrier()

    for layer in range(N_LAYERS):
        def edge_pair(p, _):
            off0 = tid * EPT + (2 * p) * CH
            pltpu.sync_copy(col_hbm.at[pl.ds(off0, CH)], colbuf0)
            _adj(colbuf0)
            g0 = pltpu.async_copy(y_hbm.at[colbuf0], rows0, semg0)
            pltpu.sync_copy(row_hbm.at[pl.ds(off0, CH)], rowbuf0)
            pltpu.sync_copy(col_hbm.at[pl.ds(off0 + CH, CH)], colbuf1)
            _adj(colbuf1)
            g1 = pltpu.async_copy(y_hbm.at[colbuf1], rows1, semg1)
            pltpu.sync_copy(row_hbm.at[pl.ds(off0 + CH, CH)], rowbuf1)
            g0.wait()
            s0 = pltpu.async_copy(rows0, acc.at[rowbuf0], sems0, add=True)
            g1.wait()
            s1 = pltpu.async_copy(rows1, acc.at[rowbuf1], sems1, add=True)
            s0.wait()
            s1.wait()
            return 0

        lax.fori_loop(0, NPAIR, edge_pair, 0)
        offt2 = tid * EPT + (NCHUNK - 1) * CH
        pltpu.sync_copy(col_hbm.at[pl.ds(offt2, CH)], colbuf0)
        _adj(colbuf0)
        pltpu.async_copy(y_hbm.at[colbuf0], rows0, semg0).wait()
        pltpu.sync_copy(row_hbm.at[pl.ds(offt2, CH)], rowbuf0)
        pltpu.sync_copy(rows0, acc.at[rowbuf0], add=True)
        plsc.subcore_barrier()

        last = layer == N_LAYERS - 1

        def dense_chunk(c, _):
            g = gbase + c * DCH
            gl = loc0 + c * DCH
            ra = pltpu.async_copy(acc.at[pl.ds(gl, DCH)],
                                  rows0.at[pl.ds(0, DCH)], sema)
            rb = pltpu.async_copy(
                out_hbm.at[pl.ds(gl, DCH), pl.ds(cid * DH, DH)], bbuf, semb)
            rc = pltpu.async_copy(dinv_hbm.at[pl.ds(g, DCH)],
                                  rows1.at[pl.ds(0, DCH)], semc)
            ra.wait()
            rb.wait()
            rc.wait()

            def rowstep(i, _):
                for j in range(DH // L):
                    a = rows0[i, pl.ds(j * L, L)]
                    dv = rows1[i, pl.ds(j * L, L)]
                    xn = dv * a
                    bbuf[i, pl.ds(j * L, L)] = (
                        bbuf[i, pl.ds(j * L, L)] + 0.25 * xn)
                    if not last:
                        rows0[i, pl.ds(j * L, L)] = dv * xn
                return 0

            lax.fori_loop(0, DCH, rowstep, 0)
            wb = pltpu.async_copy(
                bbuf, out_hbm.at[pl.ds(gl, DCH), pl.ds(cid * DH, DH)], semb)
            if not last:
                wa = pltpu.async_copy(rows0.at[pl.ds(0, DCH)],
                                      y_hbm.at[pl.ds(g, DCH)], sema)
                wa.wait()
            wb.wait()
            if not last:
                _zero_rows(bbuf, DCH, DH)
                pltpu.sync_copy(bbuf, acc.at[pl.ds(gl, DCH)])
            return 0

        lax.fori_loop(0, NDCH, dense_chunk, 0)
        if not last:
            plsc.subcore_barrier()


@jax.jit
def _lightgcn_sc(x0_cat, row, col):
    mesh = plsc.VectorSubcoreMesh(
        core_axis_name="c", subcore_axis_name="s",
        num_cores=NC, num_subcores=NS)
    return pl.kernel(
        _sc_body,
        out_type=jax.ShapeDtypeStruct((NNP, D), jnp.float32),
        mesh=mesh,
        compiler_params=pltpu.CompilerParams(
            needs_layout_passes=False, use_tc_tiling_on_sc=False),
        scratch_types=[
            pltpu.HBM((NC * NNP, DH), jnp.float32),
            pltpu.HBM((NC * NNP, DH), jnp.float32),
            pltpu.VMEM_SHARED((NNP, DH), jnp.float32),
            pltpu.VMEM((CH,), jnp.int32),
            pltpu.VMEM((CH,), jnp.int32),
            pltpu.VMEM((CH,), jnp.int32),
            pltpu.VMEM((CH,), jnp.int32),
            pltpu.VMEM((CH, DH), jnp.float32),
            pltpu.VMEM((CH, DH), jnp.float32),
            pltpu.VMEM((DCH, DH), jnp.float32),
            pltpu.SemaphoreType.DMA,
            pltpu.SemaphoreType.DMA,
            pltpu.SemaphoreType.DMA,
            pltpu.SemaphoreType.DMA,
            pltpu.SemaphoreType.DMA,
            pltpu.SemaphoreType.DMA,
            pltpu.SemaphoreType.DMA,
        ],
    )(x0_cat, row, col)


def kernel(user_emb, item_emb, edge_index):
    all0 = jnp.concatenate([user_emb, item_emb], axis=0)
    pad = jnp.zeros((NNP - NN, DH), jnp.float32)
    x0_cat = jnp.concatenate([all0[:, :DH], pad, all0[:, DH:], pad], axis=0)
    out = _lightgcn_sc(x0_cat, edge_index[0], edge_index[1])
    return out[:N_USERS], out[N_USERS:NN]


# speedup vs baseline: 17.7056x; 1.0008x over previous
import jax
import jax.numpy as jnp
from jax import lax
from jax.experimental import pallas as pl
from jax.experimental.pallas import tpu as pltpu
from jax.experimental.pallas import tpu_sc as plsc

N_USERS = 25000
N_ITEMS = 25000
NN = N_USERS + N_ITEMS
D = 64
DH = 32
E = 800000
N_LAYERS = 3

NC, NS, L = 2, 16, 16
EPT = E // NS
CH = 400
NCHUNK = EPT // CH
NPAIR = NCHUNK // 2
NNP = 50176
NPT = NNP // NS
DCH = 64
NDCH = NPT // DCH


def _zero_rows(ref, nrows, ncols):
    z = jnp.zeros((L,), jnp.float32)

    def body(i, _):
        for j in range(ncols // L):
            ref[i, pl.ds(j * L, L)] = z
        return 0

    lax.fori_loop(0, nrows, body, 0)


def _fill_ones(ref, nrows, ncols):
    o = jnp.ones((L,), jnp.float32)

    def body(i, _):
        for j in range(ncols // L):
            ref[i, pl.ds(j * L, L)] = o
        return 0

    lax.fori_loop(0, nrows, body, 0)


def _sc_body(x0_hbm, row_hbm, col_hbm, out_hbm,
             y_hbm, dinv_hbm, acc,
             rowbuf0, rowbuf1, colbuf0, colbuf1, rows0, rows1, bbuf,
             sema, semb, semc, semg0, semg1, sems0, sems1):
    cid = lax.axis_index("c")
    tid = lax.axis_index("s")
    loc0 = tid * NPT
    gbase = cid * NNP + loc0
    yoff = jnp.zeros((L,), jnp.int32) + cid * NNP

    def _adj(cb):
        def body(j, _):
            cb[pl.ds(j * L, L)] = cb[pl.ds(j * L, L)] + yoff
            return 0

        lax.fori_loop(0, CH // L, body, 0)

    _zero_rows(bbuf, DCH, DH)
    _fill_ones(rows0, CH, DH)

    def zfire(c, _):
        pltpu.async_copy(bbuf, acc.at[pl.ds(loc0 + c * DCH, DCH)], sema)
        return 0

    def zdrain(c, _):
        pltpu.make_async_copy(
            bbuf, acc.at[pl.ds(loc0 + c * DCH, DCH)], sema).wait()
        return 0

    with jax.named_scope("ph_zero"):
        lax.fori_loop(0, NDCH, zfire, 0)
        lax.fori_loop(0, NDCH, zdrain, 0)
        plsc.subcore_barrier()

    def hist_pair(p, _):
        off0 = tid * EPT + (2 * p) * CH
        pltpu.sync_copy(row_hbm.at[pl.ds(off0, CH)], rowbuf0)
        s0 = pltpu.async_copy(rows0, acc.at[rowbuf0], sems0, add=True)
        pltpu.sync_copy(row_hbm.at[pl.ds(off0 + CH, CH)], rowbuf1)
        s1 = pltpu.async_copy(rows0, acc.at[rowbuf1], sems1, add=True)
        s0.wait()
        s1.wait()
        return 0

    with jax.named_scope("ph_hist"):
        lax.fori_loop(0, NPAIR, hist_pair, 0)
        offt = tid * EPT + (NCHUNK - 1) * CH
        pltpu.sync_copy(row_hbm.at[pl.ds(offt, CH)], rowbuf0)
        pltpu.sync_copy(rows0, acc.at[rowbuf0], add=True)
        plsc.subcore_barrier()

    magic = jnp.zeros((L,), jnp.int32) + 0x5F3759DF

    def init_chunk(c, _):
        g = gbase + c * DCH
        gl = loc0 + c * DCH
        ra = pltpu.async_copy(acc.at[pl.ds(gl, DCH)], rows1.at[pl.ds(0, DCH)],
                              sema)
        rb = pltpu.async_copy(x0_hbm.at[pl.ds(g, DCH)], rows0.at[pl.ds(0, DCH)],
                              semb)
        ra.wait()
        rb.wait()

        def rowstep(i, _):
            for j in range(DH // L):
                x = rows1[i, pl.ds(j * L, L)]
                iv = plsc.bitcast(x, jnp.int32)
                y = plsc.bitcast(magic - (iv >> 1), jnp.float32)
                for _ in range(3):
                    y = y * (1.5 - 0.5 * x * y * y)
                dv = jnp.where(x > 0.5, y, 0.0)
                rows1[i, pl.ds(j * L, L)] = dv
                vx = rows0[i, pl.ds(j * L, L)]
                bbuf[i, pl.ds(j * L, L)] = 0.25 * vx
                rows0[i, pl.ds(j * L, L)] = dv * vx
            return 0

        lax.fori_loop(0, DCH, rowstep, 0)
        wa = pltpu.async_copy(rows1.at[pl.ds(0, DCH)],
                              dinv_hbm.at[pl.ds(g, DCH)], sema)
        wb = pltpu.async_copy(
            bbuf, out_hbm.at[pl.ds(gl, DCH), pl.ds(cid * DH, DH)], semb)
        wc = pltpu.async_copy(rows0.at[pl.ds(0, DCH)],
                              y_hbm.at[pl.ds(g, DCH)], semc)
        wa.wait()
        wb.wait()
        wc.wait()
        _zero_rows(bbuf, DCH, DH)
        pltpu.sync_copy(bbuf, acc.at[pl.ds(gl, DCH)])
        return 0

    with jax.named_scope("ph_init"):
        lax.fori_loop(0, NDCH, init_chunk, 0)
        plsc.subcore_barrier()

    for layer in range(N_LAYERS):
        def edge_pair(p, _):
            off0 = tid * EPT + (2 * p) * CH
            pltpu.sync_copy(col_hbm.at[pl.ds(off0, CH)], colbuf0)
            _adj(colbuf0)
            g0 = pltpu.async_copy(y_hbm.at[colbuf0], rows0, semg0)
            pltpu.sync_copy(row_hbm.at[pl.ds(off0, CH)], rowbuf0)
            pltpu.sync_copy(col_hbm.at[pl.ds(off0 + CH, CH)], colbuf1)
            _adj(colbuf1)
            g1 = pltpu.async_copy(y_hbm.at[colbuf1], rows1, semg1)
            pltpu.sync_copy(row_hbm.at[pl.ds(off0 + CH, CH)], rowbuf1)
            g0.wait()
            s0 = pltpu.async_copy(rows0, acc.at[rowbuf0], sems0, add=True)
            g1.wait()
            s1 = pltpu.async_copy(rows1, acc.at[rowbuf1], sems1, add=True)
            s0.wait()
            s1.wait()
            return 0

        ns_e = jax.named_scope(f"ph_edge{layer}")
        ns_e.__enter__()
        lax.fori_loop(0, NPAIR, edge_pair, 0)
        offt2 = tid * EPT + (NCHUNK - 1) * CH
        pltpu.sync_copy(col_hbm.at[pl.ds(offt2, CH)], colbuf0)
        _adj(colbuf0)
        pltpu.async_copy(y_hbm.at[colbuf0], rows0, semg0).wait()
        pltpu.sync_copy(row_hbm.at[pl.ds(offt2, CH)], rowbuf0)
        pltpu.sync_copy(rows0, acc.at[rowbuf0], add=True)
        plsc.subcore_barrier()
        ns_e.__exit__(None, None, None)

        last = layer == N_LAYERS - 1

        def dense_chunk(c, _):
            g = gbase + c * DCH
            gl = loc0 + c * DCH
            ra = pltpu.async_copy(acc.at[pl.ds(gl, DCH)],
                                  rows0.at[pl.ds(0, DCH)], sema)
            rb = pltpu.async_copy(
                out_hbm.at[pl.ds(gl, DCH), pl.ds(cid * DH, DH)], bbuf, semb)
            rc = pltpu.async_copy(dinv_hbm.at[pl.ds(g, DCH)],
                                  rows1.at[pl.ds(0, DCH)], semc)
            ra.wait()
            rb.wait()
            rc.wait()

            def rowstep(i, _):
                for j in range(DH // L):
                    a = rows0[i, pl.ds(j * L, L)]
                    dv = rows1[i, pl.ds(j * L, L)]
                    xn = dv * a
                    bbuf[i, pl.ds(j * L, L)] = (
                        bbuf[i, pl.ds(j * L, L)] + 0.25 * xn)
                    if not last:
                        rows0[i, pl.ds(j * L, L)] = dv * xn
                return 0

            lax.fori_loop(0, DCH, rowstep, 0)
            wb = pltpu.async_copy(
                bbuf, out_hbm.at[pl.ds(gl, DCH), pl.ds(cid * DH, DH)], semb)
            if not last:
                wa = pltpu.async_copy(rows0.at[pl.ds(0, DCH)],
                                      y_hbm.at[pl.ds(g, DCH)], sema)
                wa.wait()
            wb.wait()
            if not last:
                _zero_rows(bbuf, DCH, DH)
                pltpu.sync_copy(bbuf, acc.at[pl.ds(gl, DCH)])
            return 0

        with jax.named_scope(f"ph_dense{layer}"):
            lax.fori_loop(0, NDCH, dense_chunk, 0)
            if not last:
                plsc.subcore_barrier()


@jax.jit
def _lightgcn_sc(x0_cat, row, col):
    mesh = plsc.VectorSubcoreMesh(
        core_axis_name="c", subcore_axis_name="s",
        num_cores=NC, num_subcores=NS)
    return pl.kernel(
        _sc_body,
        out_type=jax.ShapeDtypeStruct((NNP, D), jnp.float32),
        mesh=mesh,
        compiler_params=pltpu.CompilerParams(
            needs_layout_passes=False, use_tc_tiling_on_sc=False),
        scratch_types=[
            pltpu.HBM((NC * NNP, DH), jnp.float32),
            pltpu.HBM((NC * NNP, DH), jnp.float32),
            pltpu.VMEM_SHARED((NNP, DH), jnp.float32),
            pltpu.VMEM((CH,), jnp.int32),
            pltpu.VMEM((CH,), jnp.int32),
            pltpu.VMEM((CH,), jnp.int32),
            pltpu.VMEM((CH,), jnp.int32),
            pltpu.VMEM((CH, DH), jnp.float32),
            pltpu.VMEM((CH, DH), jnp.float32),
            pltpu.VMEM((DCH, DH), jnp.float32),
            pltpu.SemaphoreType.DMA,
            pltpu.SemaphoreType.DMA,
            pltpu.SemaphoreType.DMA,
            pltpu.SemaphoreType.DMA,
            pltpu.SemaphoreType.DMA,
            pltpu.SemaphoreType.DMA,
            pltpu.SemaphoreType.DMA,
        ],
    )(x0_cat, row, col)


def kernel(user_emb, item_emb, edge_index):
    all0 = jnp.concatenate([user_emb, item_emb], axis=0)
    pad = jnp.zeros((NNP - NN, DH), jnp.float32)
    x0_cat = jnp.concatenate([all0[:, :DH], pad, all0[:, DH:], pad], axis=0)
    out = _lightgcn_sc(x0_cat, edge_index[0], edge_index[1])
    return out[:N_USERS], out[N_USERS:NN]


# speedup vs baseline: 18.3839x; 1.0383x over previous
import jax
import jax.numpy as jnp
from jax import lax
from jax.experimental import pallas as pl
from jax.experimental.pallas import tpu as pltpu
from jax.experimental.pallas import tpu_sc as plsc

N_USERS = 25000
N_ITEMS = 25000
NN = N_USERS + N_ITEMS
D = 64
DH = 32
E = 800000
N_LAYERS = 3

NC, NS, L = 2, 16, 16
EPT = E // NS
CH = 400
NCHUNK = EPT // CH
NPAIR = NCHUNK // 2
NNP = 50176
NPT = NNP // NS
DCH = 64
NDCH = NPT // DCH


def _zero_rows(ref, nrows, ncols):
    z = jnp.zeros((L,), jnp.float32)

    def body(i, _):
        for j in range(ncols // L):
            ref[i, pl.ds(j * L, L)] = z
        return 0

    lax.fori_loop(0, nrows, body, 0)


def _fill_ones(ref, nrows, ncols):
    o = jnp.ones((L,), jnp.float32)

    def body(i, _):
        for j in range(ncols // L):
            ref[i, pl.ds(j * L, L)] = o
        return 0

    lax.fori_loop(0, nrows, body, 0)


def _sc_body(x0_hbm, edge_hbm, out_hbm,
             y_hbm, dinv_hbm, acc,
             rowbuf0, rowbuf1, colbuf0, colbuf1, rows0, rows1, bbuf,
             sema, semb, semc, semg0, semg1, sems0, sems1):
    cid = lax.axis_index("c")
    tid = lax.axis_index("s")
    loc0 = tid * NPT
    gbase = cid * NNP + loc0
    yoff = jnp.zeros((L,), jnp.int32) + cid * NNP

    def _adj(cb):
        def body(j, _):
            cb[pl.ds(j * L, L)] = cb[pl.ds(j * L, L)] + yoff
            return 0

        lax.fori_loop(0, CH // L, body, 0)

    _zero_rows(bbuf, DCH, DH)
    _fill_ones(rows0, CH, DH)

    def zfire(c, _):
        pltpu.async_copy(bbuf, acc.at[pl.ds(loc0 + c * DCH, DCH)], sema)
        return 0

    def zdrain(c, _):
        pltpu.make_async_copy(
            bbuf, acc.at[pl.ds(loc0 + c * DCH, DCH)], sema).wait()
        return 0

    with jax.named_scope("ph_zero"):
        lax.fori_loop(0, NDCH, zfire, 0)
        lax.fori_loop(0, NDCH, zdrain, 0)
        plsc.subcore_barrier()

    def hist_pair(p, _):
        off0 = tid * EPT + (2 * p) * CH
        pltpu.sync_copy(edge_hbm.at[0, pl.ds(off0, CH)], rowbuf0)
        s0 = pltpu.async_copy(rows0, acc.at[rowbuf0], sems0, add=True)
        pltpu.sync_copy(edge_hbm.at[0, pl.ds(off0 + CH, CH)], rowbuf1)
        s1 = pltpu.async_copy(rows0, acc.at[rowbuf1], sems1, add=True)
        s0.wait()
        s1.wait()
        return 0

    with jax.named_scope("ph_hist"):
        lax.fori_loop(0, NPAIR, hist_pair, 0)
        offt = tid * EPT + (NCHUNK - 1) * CH
        pltpu.sync_copy(edge_hbm.at[0, pl.ds(offt, CH)], rowbuf0)
        pltpu.sync_copy(rows0, acc.at[rowbuf0], add=True)
        plsc.subcore_barrier()

    magic = jnp.zeros((L,), jnp.int32) + 0x5F3759DF

    def init_chunk(c, _):
        g = gbase + c * DCH
        gl = loc0 + c * DCH
        ra = pltpu.async_copy(acc.at[pl.ds(gl, DCH)], rows1.at[pl.ds(0, DCH)],
                              sema)
        rb = pltpu.async_copy(
            x0_hbm.at[pl.ds(gl, DCH), pl.ds(cid * DH, DH)],
            rows0.at[pl.ds(0, DCH)], semb)
        ra.wait()
        rb.wait()

        def rowstep(i, _):
            for j in range(DH // L):
                x = rows1[i, pl.ds(j * L, L)]
                iv = plsc.bitcast(x, jnp.int32)
                y = plsc.bitcast(magic - (iv >> 1), jnp.float32)
                for _ in range(3):
                    y = y * (1.5 - 0.5 * x * y * y)
                dv = jnp.where(x > 0.5, y, 0.0)
                rows1[i, pl.ds(j * L, L)] = dv
                vx = rows0[i, pl.ds(j * L, L)]
                bbuf[i, pl.ds(j * L, L)] = 0.25 * vx
                rows0[i, pl.ds(j * L, L)] = dv * vx
            return 0

        lax.fori_loop(0, DCH, rowstep, 0)
        wa = pltpu.async_copy(rows1.at[pl.ds(0, DCH)],
                              dinv_hbm.at[pl.ds(g, DCH)], sema)
        wb = pltpu.async_copy(
            bbuf, out_hbm.at[pl.ds(gl, DCH), pl.ds(cid * DH, DH)], semb)
        wc = pltpu.async_copy(rows0.at[pl.ds(0, DCH)],
                              y_hbm.at[pl.ds(g, DCH)], semc)
        wa.wait()
        wb.wait()
        wc.wait()
        _zero_rows(bbuf, DCH, DH)
        pltpu.sync_copy(bbuf, acc.at[pl.ds(gl, DCH)])
        return 0

    with jax.named_scope("ph_init"):
        lax.fori_loop(0, NDCH, init_chunk, 0)
        plsc.subcore_barrier()

    for layer in range(N_LAYERS):
        def edge_pair(p, _):
            off0 = tid * EPT + (2 * p) * CH
            pltpu.sync_copy(edge_hbm.at[1, pl.ds(off0, CH)], colbuf0)
            _adj(colbuf0)
            g0 = pltpu.async_copy(y_hbm.at[colbuf0], rows0, semg0)
            pltpu.sync_copy(edge_hbm.at[0, pl.ds(off0, CH)], rowbuf0)
            pltpu.sync_copy(edge_hbm.at[1, pl.ds(off0 + CH, CH)], colbuf1)
            _adj(colbuf1)
            g1 = pltpu.async_copy(y_hbm.at[colbuf1], rows1, semg1)
            pltpu.sync_copy(edge_hbm.at[0, pl.ds(off0 + CH, CH)], rowbuf1)
            g0.wait()
            s0 = pltpu.async_copy(rows0, acc.at[rowbuf0], sems0, add=True)
            g1.wait()
            s1 = pltpu.async_copy(rows1, acc.at[rowbuf1], sems1, add=True)
            s0.wait()
            s1.wait()
            return 0

        ns_e = jax.named_scope(f"ph_edge{layer}")
        ns_e.__enter__()
        lax.fori_loop(0, NPAIR, edge_pair, 0)
        offt2 = tid * EPT + (NCHUNK - 1) * CH
        pltpu.sync_copy(edge_hbm.at[1, pl.ds(offt2, CH)], colbuf0)
        _adj(colbuf0)
        pltpu.async_copy(y_hbm.at[colbuf0], rows0, semg0).wait()
        pltpu.sync_copy(edge_hbm.at[0, pl.ds(offt2, CH)], rowbuf0)
        pltpu.sync_copy(rows0, acc.at[rowbuf0], add=True)
        plsc.subcore_barrier()
        ns_e.__exit__(None, None, None)

        last = layer == N_LAYERS - 1

        def dense_chunk(c, _):
            g = gbase + c * DCH
            gl = loc0 + c * DCH
            ra = pltpu.async_copy(acc.at[pl.ds(gl, DCH)],
                                  rows0.at[pl.ds(0, DCH)], sema)
            rb = pltpu.async_copy(
                out_hbm.at[pl.ds(gl, DCH), pl.ds(cid * DH, DH)], bbuf, semb)
            rc = pltpu.async_copy(dinv_hbm.at[pl.ds(g, DCH)],
                                  rows1.at[pl.ds(0, DCH)], semc)
            ra.wait()
            rb.wait()
            rc.wait()

            def rowstep(i, _):
                for j in range(DH // L):
                    a = rows0[i, pl.ds(j * L, L)]
                    dv = rows1[i, pl.ds(j * L, L)]
                    xn = dv * a
                    bbuf[i, pl.ds(j * L, L)] = (
                        bbuf[i, pl.ds(j * L, L)] + 0.25 * xn)
                    if not last:
                        rows0[i, pl.ds(j * L, L)] = dv * xn
                return 0

            lax.fori_loop(0, DCH, rowstep, 0)
            wb = pltpu.async_copy(
                bbuf, out_hbm.at[pl.ds(gl, DCH), pl.ds(cid * DH, DH)], semb)
            if not last:
                wa = pltpu.async_copy(rows0.at[pl.ds(0, DCH)],
                                      y_hbm.at[pl.ds(g, DCH)], sema)
                wa.wait()
            wb.wait()
            if not last:
                _zero_rows(bbuf, DCH, DH)
                pltpu.sync_copy(bbuf, acc.at[pl.ds(gl, DCH)])
            return 0

        with jax.named_scope(f"ph_dense{layer}"):
            lax.fori_loop(0, NDCH, dense_chunk, 0)
            if not last:
                plsc.subcore_barrier()


@jax.jit
def _lightgcn_sc(x0_cat, edge_index):
    mesh = plsc.VectorSubcoreMesh(
        core_axis_name="c", subcore_axis_name="s",
        num_cores=NC, num_subcores=NS)
    return pl.kernel(
        _sc_body,
        out_type=jax.ShapeDtypeStruct((NNP, D), jnp.float32),
        mesh=mesh,
        compiler_params=pltpu.CompilerParams(
            needs_layout_passes=False, use_tc_tiling_on_sc=False),
        scratch_types=[
            pltpu.HBM((NC * NNP, DH), jnp.float32),
            pltpu.HBM((NC * NNP, DH), jnp.float32),
            pltpu.VMEM_SHARED((NNP, DH), jnp.float32),
            pltpu.VMEM((CH,), jnp.int32),
            pltpu.VMEM((CH,), jnp.int32),
            pltpu.VMEM((CH,), jnp.int32),
            pltpu.VMEM((CH,), jnp.int32),
            pltpu.VMEM((CH, DH), jnp.float32),
            pltpu.VMEM((CH, DH), jnp.float32),
            pltpu.VMEM((DCH, DH), jnp.float32),
            pltpu.SemaphoreType.DMA,
            pltpu.SemaphoreType.DMA,
            pltpu.SemaphoreType.DMA,
            pltpu.SemaphoreType.DMA,
            pltpu.SemaphoreType.DMA,
            pltpu.SemaphoreType.DMA,
            pltpu.SemaphoreType.DMA,
        ],
    )(x0_cat, edge_index)


def kernel(user_emb, item_emb, edge_index):
    pad = jnp.zeros((NNP - NN, D), jnp.float32)
    x0_cat = jnp.concatenate([user_emb, item_emb, pad], axis=0)
    out = _lightgcn_sc(x0_cat, edge_index)
    return out[:N_USERS], out[N_USERS:NN]


# speedup vs baseline: 18.8459x; 1.0251x over previous
import jax
import jax.numpy as jnp
from jax import lax
from jax.experimental import pallas as pl
from jax.experimental.pallas import tpu as pltpu
from jax.experimental.pallas import tpu_sc as plsc

N_USERS = 25000
N_ITEMS = 25000
NN = N_USERS + N_ITEMS
D = 64
DH = 32
E = 800000
N_LAYERS = 3

NC, NS, L = 2, 16, 16
EPT = E // NS
CH = 400
NCHUNK = EPT // CH
NPAIR = NCHUNK // 2
NNP = 50176
NPT = NNP // NS
DCH = 64
NDCH = NPT // DCH


def _zero_rows(ref, nrows, ncols):
    z = jnp.zeros((L,), jnp.float32)

    def body(i, _):
        for j in range(ncols // L):
            ref[i, pl.ds(j * L, L)] = z
        return 0

    lax.fori_loop(0, nrows, body, 0)


def _fill_ones(ref, nrows, ncols):
    o = jnp.ones((L,), jnp.float32)

    def body(i, _):
        for j in range(ncols // L):
            ref[i, pl.ds(j * L, L)] = o
        return 0

    lax.fori_loop(0, nrows, body, 0)


def _sc_body(x0_hbm, edge_hbm, out_hbm,
             y_hbm, dinv_hbm, acc,
             rowbuf0, rowbuf1, colbuf0, colbuf1, rows0, rows1, bbuf,
             sema, semb, semc, semd, semg0, semg1, sems0, sems1):
    cid = lax.axis_index("c")
    tid = lax.axis_index("s")
    loc0 = tid * NPT
    gbase = cid * NNP + loc0
    yoff = jnp.zeros((L,), jnp.int32) + cid * NNP

    def _adj(cb):
        def body(j, _):
            cb[pl.ds(j * L, L)] = cb[pl.ds(j * L, L)] + yoff
            return 0

        lax.fori_loop(0, CH // L, body, 0)

    _zero_rows(bbuf, DCH, DH)
    _fill_ones(rows0, CH, DH)

    def zfire(c, _):
        pltpu.async_copy(bbuf, acc.at[pl.ds(loc0 + c * DCH, DCH)], sema)
        return 0

    def zdrain(c, _):
        pltpu.make_async_copy(
            bbuf, acc.at[pl.ds(loc0 + c * DCH, DCH)], sema).wait()
        return 0

    with jax.named_scope("ph_zero"):
        lax.fori_loop(0, NDCH, zfire, 0)
        lax.fori_loop(0, NDCH, zdrain, 0)
        plsc.subcore_barrier()

    def hist_pair(p, _):
        off0 = tid * EPT + (2 * p) * CH
        rr0 = pltpu.async_copy(edge_hbm.at[0, pl.ds(off0, CH)], rowbuf0, sema)
        rr1 = pltpu.async_copy(
            edge_hbm.at[0, pl.ds(off0 + CH, CH)], rowbuf1, semb)
        rr0.wait()
        s0 = pltpu.async_copy(rows0, acc.at[rowbuf0], sems0, add=True)
        rr1.wait()
        s1 = pltpu.async_copy(rows0, acc.at[rowbuf1], sems1, add=True)
        s0.wait()
        s1.wait()
        return 0

    with jax.named_scope("ph_hist"):
        lax.fori_loop(0, NPAIR, hist_pair, 0)
        offt = tid * EPT + (NCHUNK - 1) * CH
        pltpu.sync_copy(edge_hbm.at[0, pl.ds(offt, CH)], rowbuf0)
        pltpu.sync_copy(rows0, acc.at[rowbuf0], add=True)
        plsc.subcore_barrier()

    magic = jnp.zeros((L,), jnp.int32) + 0x5F3759DF

    def init_chunk(c, _):
        g = gbase + c * DCH
        gl = loc0 + c * DCH
        ra = pltpu.async_copy(acc.at[pl.ds(gl, DCH)], rows1.at[pl.ds(0, DCH)],
                              sema)
        rb = pltpu.async_copy(
            x0_hbm.at[pl.ds(gl, DCH), pl.ds(cid * DH, DH)],
            rows0.at[pl.ds(0, DCH)], semb)
        ra.wait()
        rb.wait()

        def rowstep(i, _):
            for j in range(DH // L):
                x = rows1[i, pl.ds(j * L, L)]
                iv = plsc.bitcast(x, jnp.int32)
                y = plsc.bitcast(magic - (iv >> 1), jnp.float32)
                for _ in range(3):
                    y = y * (1.5 - 0.5 * x * y * y)
                dv = jnp.where(x > 0.5, y, 0.0)
                rows1[i, pl.ds(j * L, L)] = dv
                vx = rows0[i, pl.ds(j * L, L)]
                bbuf[i, pl.ds(j * L, L)] = 0.25 * vx
                rows0[i, pl.ds(j * L, L)] = dv * vx
            return 0

        lax.fori_loop(0, DCH, rowstep, 0)
        wa = pltpu.async_copy(rows1.at[pl.ds(0, DCH)],
                              dinv_hbm.at[pl.ds(g, DCH)], sema)
        wb = pltpu.async_copy(
            bbuf, out_hbm.at[pl.ds(gl, DCH), pl.ds(cid * DH, DH)], semb)
        wc = pltpu.async_copy(rows0.at[pl.ds(0, DCH)],
                              y_hbm.at[pl.ds(g, DCH)], semc)
        wa.wait()
        wb.wait()
        wc.wait()
        _zero_rows(bbuf, DCH, DH)
        pltpu.sync_copy(bbuf, acc.at[pl.ds(gl, DCH)])
        return 0

    with jax.named_scope("ph_init"):
        lax.fori_loop(0, NDCH, init_chunk, 0)
        plsc.subcore_barrier()

    for layer in range(N_LAYERS):
        def edge_pair(p, _):
            off0 = tid * EPT + (2 * p) * CH
            rc0 = pltpu.async_copy(
                edge_hbm.at[1, pl.ds(off0, CH)], colbuf0, semc)
            rc1 = pltpu.async_copy(
                edge_hbm.at[1, pl.ds(off0 + CH, CH)], colbuf1, semd)
            rr0 = pltpu.async_copy(
                edge_hbm.at[0, pl.ds(off0, CH)], rowbuf0, sema)
            rr1 = pltpu.async_copy(
                edge_hbm.at[0, pl.ds(off0 + CH, CH)], rowbuf1, semb)
            rc0.wait()
            _adj(colbuf0)
            g0 = pltpu.async_copy(y_hbm.at[colbuf0], rows0, semg0)
            rc1.wait()
            _adj(colbuf1)
            g1 = pltpu.async_copy(y_hbm.at[colbuf1], rows1, semg1)
            g0.wait()
            rr0.wait()
            s0 = pltpu.async_copy(rows0, acc.at[rowbuf0], sems0, add=True)
            g1.wait()
            rr1.wait()
            s1 = pltpu.async_copy(rows1, acc.at[rowbuf1], sems1, add=True)
            s0.wait()
            s1.wait()
            return 0

        ns_e = jax.named_scope(f"ph_edge{layer}")
        ns_e.__enter__()
        lax.fori_loop(0, NPAIR, edge_pair, 0)
        offt2 = tid * EPT + (NCHUNK - 1) * CH
        pltpu.sync_copy(edge_hbm.at[1, pl.ds(offt2, CH)], colbuf0)
        _adj(colbuf0)
        pltpu.async_copy(y_hbm.at[colbuf0], rows0, semg0).wait()
        pltpu.sync_copy(edge_hbm.at[0, pl.ds(offt2, CH)], rowbuf0)
        pltpu.sync_copy(rows0, acc.at[rowbuf0], add=True)
        plsc.subcore_barrier()
        ns_e.__exit__(None, None, None)

        last = layer == N_LAYERS - 1

        def dense_chunk(c, _):
            g = gbase + c * DCH
            gl = loc0 + c * DCH
            ra = pltpu.async_copy(acc.at[pl.ds(gl, DCH)],
                                  rows0.at[pl.ds(0, DCH)], sema)
            rb = pltpu.async_copy(
                out_hbm.at[pl.ds(gl, DCH), pl.ds(cid * DH, DH)], bbuf, semb)
            rc = pltpu.async_copy(dinv_hbm.at[pl.ds(g, DCH)],
                                  rows1.at[pl.ds(0, DCH)], semc)
            ra.wait()
            rb.wait()
            rc.wait()

            def rowstep(i, _):
                for j in range(DH // L):
                    a = rows0[i, pl.ds(j * L, L)]
                    dv = rows1[i, pl.ds(j * L, L)]
                    xn = dv * a
                    bbuf[i, pl.ds(j * L, L)] = (
                        bbuf[i, pl.ds(j * L, L)] + 0.25 * xn)
                    if not last:
                        rows0[i, pl.ds(j * L, L)] = dv * xn
                return 0

            lax.fori_loop(0, DCH, rowstep, 0)
            wb = pltpu.async_copy(
                bbuf, out_hbm.at[pl.ds(gl, DCH), pl.ds(cid * DH, DH)], semb)
            if not last:
                wa = pltpu.async_copy(rows0.at[pl.ds(0, DCH)],
                                      y_hbm.at[pl.ds(g, DCH)], sema)
                wa.wait()
            wb.wait()
            if not last:
                _zero_rows(bbuf, DCH, DH)
                pltpu.sync_copy(bbuf, acc.at[pl.ds(gl, DCH)])
            return 0

        with jax.named_scope(f"ph_dense{layer}"):
            lax.fori_loop(0, NDCH, dense_chunk, 0)
            if not last:
                plsc.subcore_barrier()


@jax.jit
def _lightgcn_sc(x0_cat, edge_index):
    mesh = plsc.VectorSubcoreMesh(
        core_axis_name="c", subcore_axis_name="s",
        num_cores=NC, num_subcores=NS)
    return pl.kernel(
        _sc_body,
        out_type=jax.ShapeDtypeStruct((NNP, D), jnp.float32),
        mesh=mesh,
        compiler_params=pltpu.CompilerParams(
            needs_layout_passes=False, use_tc_tiling_on_sc=False),
        scratch_types=[
            pltpu.HBM((NC * NNP, DH), jnp.float32),
            pltpu.HBM((NC * NNP, DH), jnp.float32),
            pltpu.VMEM_SHARED((NNP, DH), jnp.float32),
            pltpu.VMEM((CH,), jnp.int32),
            pltpu.VMEM((CH,), jnp.int32),
            pltpu.VMEM((CH,), jnp.int32),
            pltpu.VMEM((CH,), jnp.int32),
            pltpu.VMEM((CH, DH), jnp.float32),
            pltpu.VMEM((CH, DH), jnp.float32),
            pltpu.VMEM((DCH, DH), jnp.float32),
            pltpu.SemaphoreType.DMA,
            pltpu.SemaphoreType.DMA,
            pltpu.SemaphoreType.DMA,
            pltpu.SemaphoreType.DMA,
            pltpu.SemaphoreType.DMA,
            pltpu.SemaphoreType.DMA,
            pltpu.SemaphoreType.DMA,
            pltpu.SemaphoreType.DMA,
        ],
    )(x0_cat, edge_index)


def kernel(user_emb, item_emb, edge_index):
    pad = jnp.zeros((NNP - NN, D), jnp.float32)
    x0_cat = jnp.concatenate([user_emb, item_emb, pad], axis=0)
    out = _lightgcn_sc(x0_cat, edge_index)
    return out[:N_USERS], out[N_USERS:NN]


# speedup vs baseline: 18.8857x; 1.0021x over previous
import jax
import jax.numpy as jnp
from jax import lax
from jax.experimental import pallas as pl
from jax.experimental.pallas import tpu as pltpu
from jax.experimental.pallas import tpu_sc as plsc

N_USERS = 25000
N_ITEMS = 25000
NN = N_USERS + N_ITEMS
D = 64
DH = 32
E = 800000
N_LAYERS = 3

NC, NS, L = 2, 16, 16
EPT = E // NS
CH = 400
NCHUNK = EPT // CH
NPAIR = NCHUNK // 2
NNP = 50176
NPT = NNP // NS
DCH = 64
NDCH = NPT // DCH


def _zero_rows(ref, nrows, ncols):
    z = jnp.zeros((L,), jnp.float32)

    def body(i, _):
        for j in range(ncols // L):
            ref[i, pl.ds(j * L, L)] = z
        return 0

    lax.fori_loop(0, nrows, body, 0)


def _fill_ones(ref, nrows, ncols):
    o = jnp.ones((L,), jnp.float32)

    def body(i, _):
        for j in range(ncols // L):
            ref[i, pl.ds(j * L, L)] = o
        return 0

    lax.fori_loop(0, nrows, body, 0)


def _sc_body(x0_hbm, edge_hbm, out_hbm,
             y_hbm, dinv_hbm, acc,
             rowbuf0, rowbuf1, colbuf0, colbuf1, rows0, rows1, bbuf,
             sema, semb, semc, semd, semg0, semg1, sems0, sems1):
    cid = lax.axis_index("c")
    tid = lax.axis_index("s")
    loc0 = tid * NPT
    gbase = cid * NNP + loc0
    yoff = jnp.zeros((L,), jnp.int32) + cid * NNP

    def _adj(cb):
        def body(j, _):
            cb[pl.ds(j * L, L)] = cb[pl.ds(j * L, L)] + yoff
            return 0

        lax.fori_loop(0, CH // L, body, 0)

    _zero_rows(bbuf, DCH, DH)
    _fill_ones(rows0, CH, DH)

    def zfire(c, _):
        pltpu.async_copy(bbuf, acc.at[pl.ds(loc0 + c * DCH, DCH)], sema)
        return 0

    def zdrain(c, _):
        pltpu.make_async_copy(
            bbuf, acc.at[pl.ds(loc0 + c * DCH, DCH)], sema).wait()
        return 0

    with jax.named_scope("ph_zero"):
        lax.fori_loop(0, NDCH, zfire, 0)
        lax.fori_loop(0, NDCH, zdrain, 0)
        plsc.subcore_barrier()

    def hist_pair(p, _):
        off0 = tid * EPT + (2 * p) * CH
        rr0 = pltpu.async_copy(edge_hbm.at[0, pl.ds(off0, CH)], rowbuf0, sema)
        rr1 = pltpu.async_copy(
            edge_hbm.at[0, pl.ds(off0 + CH, CH)], rowbuf1, semb)
        rr0.wait()
        s0 = pltpu.async_copy(rows0, acc.at[rowbuf0], sems0, add=True)
        rr1.wait()
        s1 = pltpu.async_copy(rows0, acc.at[rowbuf1], sems1, add=True)
        s0.wait()
        s1.wait()
        return 0

    with jax.named_scope("ph_hist"):
        lax.fori_loop(0, NPAIR, hist_pair, 0)
        offt = tid * EPT + (NCHUNK - 1) * CH
        pltpu.sync_copy(edge_hbm.at[0, pl.ds(offt, CH)], rowbuf0)
        pltpu.sync_copy(rows0, acc.at[rowbuf0], add=True)
        plsc.subcore_barrier()

    magic = jnp.zeros((L,), jnp.int32) + 0x5F3759DF

    def init_chunk(c, _):
        g = gbase + c * DCH
        gl = loc0 + c * DCH
        ra = pltpu.async_copy(acc.at[pl.ds(gl, DCH)], rows1.at[pl.ds(0, DCH)],
                              sema)
        rb = pltpu.async_copy(
            x0_hbm.at[pl.ds(gl, DCH), pl.ds(cid * DH, DH)],
            rows0.at[pl.ds(0, DCH)], semb)
        ra.wait()
        rb.wait()

        def rowstep(i, _):
            for j in range(DH // L):
                x = rows1[i, pl.ds(j * L, L)]
                iv = plsc.bitcast(x, jnp.int32)
                y = plsc.bitcast(magic - (iv >> 1), jnp.float32)
                for _ in range(3):
                    y = y * (1.5 - 0.5 * x * y * y)
                dv = jnp.where(x > 0.5, y, 0.0)
                rows1[i, pl.ds(j * L, L)] = dv
                vx = rows0[i, pl.ds(j * L, L)]
                bbuf[i, pl.ds(j * L, L)] = 0.25 * vx
                rows0[i, pl.ds(j * L, L)] = dv * vx
            return 0

        lax.fori_loop(0, DCH, rowstep, 0)
        wa = pltpu.async_copy(rows1.at[pl.ds(0, DCH)],
                              dinv_hbm.at[pl.ds(g, DCH)], sema)
        wb = pltpu.async_copy(
            bbuf, out_hbm.at[pl.ds(gl, DCH), pl.ds(cid * DH, DH)], semb)
        wc = pltpu.async_copy(rows0.at[pl.ds(0, DCH)],
                              y_hbm.at[pl.ds(g, DCH)], semc)
        wa.wait()
        wb.wait()
        wc.wait()
        _zero_rows(bbuf, DCH, DH)
        pltpu.sync_copy(bbuf, acc.at[pl.ds(gl, DCH)])
        return 0

    with jax.named_scope("ph_init"):
        lax.fori_loop(0, NDCH, init_chunk, 0)
        plsc.subcore_barrier()

    for layer in range(N_LAYERS):
        def edge_pair(p, _):
            off0 = tid * EPT + (2 * p) * CH
            rc0 = pltpu.async_copy(
                edge_hbm.at[1, pl.ds(off0, CH)], colbuf0, semc)
            rc1 = pltpu.async_copy(
                edge_hbm.at[1, pl.ds(off0 + CH, CH)], colbuf1, semd)
            rr0 = pltpu.async_copy(
                edge_hbm.at[0, pl.ds(off0, CH)], rowbuf0, sema)
            rr1 = pltpu.async_copy(
                edge_hbm.at[0, pl.ds(off0 + CH, CH)], rowbuf1, semb)
            rc0.wait()
            g0 = pltpu.async_copy(
                y_hbm.at[pl.ds(cid * NNP, NNP)].at[colbuf0], rows0, semg0)
            rc1.wait()
            g1 = pltpu.async_copy(
                y_hbm.at[pl.ds(cid * NNP, NNP)].at[colbuf1], rows1, semg1)
            g0.wait()
            rr0.wait()
            s0 = pltpu.async_copy(rows0, acc.at[rowbuf0], sems0, add=True)
            g1.wait()
            rr1.wait()
            s1 = pltpu.async_copy(rows1, acc.at[rowbuf1], sems1, add=True)
            s0.wait()
            s1.wait()
            return 0

        ns_e = jax.named_scope(f"ph_edge{layer}")
        ns_e.__enter__()
        lax.fori_loop(0, NPAIR, edge_pair, 0)
        offt2 = tid * EPT + (NCHUNK - 1) * CH
        pltpu.sync_copy(edge_hbm.at[1, pl.ds(offt2, CH)], colbuf0)
        pltpu.async_copy(
            y_hbm.at[pl.ds(cid * NNP, NNP)].at[colbuf0], rows0, semg0).wait()
        pltpu.sync_copy(edge_hbm.at[0, pl.ds(offt2, CH)], rowbuf0)
        pltpu.sync_copy(rows0, acc.at[rowbuf0], add=True)
        plsc.subcore_barrier()
        ns_e.__exit__(None, None, None)

        last = layer == N_LAYERS - 1

        def dense_chunk(c, _):
            g = gbase + c * DCH
            gl = loc0 + c * DCH
            ra = pltpu.async_copy(acc.at[pl.ds(gl, DCH)],
                                  rows0.at[pl.ds(0, DCH)], sema)
            rb = pltpu.async_copy(
                out_hbm.at[pl.ds(gl, DCH), pl.ds(cid * DH, DH)], bbuf, semb)
            rc = pltpu.async_copy(dinv_hbm.at[pl.ds(g, DCH)],
                                  rows1.at[pl.ds(0, DCH)], semc)
            ra.wait()
            rb.wait()
            rc.wait()

            def rowstep(i, _):
                for j in range(DH // L):
                    a = rows0[i, pl.ds(j * L, L)]
                    dv = rows1[i, pl.ds(j * L, L)]
                    xn = dv * a
                    bbuf[i, pl.ds(j * L, L)] = (
                        bbuf[i, pl.ds(j * L, L)] + 0.25 * xn)
                    if not last:
                        rows0[i, pl.ds(j * L, L)] = dv * xn
                return 0

            lax.fori_loop(0, DCH, rowstep, 0)
            wb = pltpu.async_copy(
                bbuf, out_hbm.at[pl.ds(gl, DCH), pl.ds(cid * DH, DH)], semb)
            if not last:
                wa = pltpu.async_copy(rows0.at[pl.ds(0, DCH)],
                                      y_hbm.at[pl.ds(g, DCH)], sema)
                wa.wait()
            wb.wait()
            if not last:
                _zero_rows(bbuf, DCH, DH)
                pltpu.sync_copy(bbuf, acc.at[pl.ds(gl, DCH)])
            return 0

        with jax.named_scope(f"ph_dense{layer}"):
            lax.fori_loop(0, NDCH, dense_chunk, 0)
            if not last:
                plsc.subcore_barrier()


@jax.jit
def _lightgcn_sc(x0_cat, edge_index):
    mesh = plsc.VectorSubcoreMesh(
        core_axis_name="c", subcore_axis_name="s",
        num_cores=NC, num_subcores=NS)
    return pl.kernel(
        _sc_body,
        out_type=jax.ShapeDtypeStruct((NNP, D), jnp.float32),
        mesh=mesh,
        compiler_params=pltpu.CompilerParams(
            needs_layout_passes=False, use_tc_tiling_on_sc=False),
        scratch_types=[
            pltpu.HBM((NC * NNP, DH), jnp.float32),
            pltpu.HBM((NC * NNP, DH), jnp.float32),
            pltpu.VMEM_SHARED((NNP, DH), jnp.float32),
            pltpu.VMEM((CH,), jnp.int32),
            pltpu.VMEM((CH,), jnp.int32),
            pltpu.VMEM((CH,), jnp.int32),
            pltpu.VMEM((CH,), jnp.int32),
            pltpu.VMEM((CH, DH), jnp.float32),
            pltpu.VMEM((CH, DH), jnp.float32),
            pltpu.VMEM((DCH, DH), jnp.float32),
            pltpu.SemaphoreType.DMA,
            pltpu.SemaphoreType.DMA,
            pltpu.SemaphoreType.DMA,
            pltpu.SemaphoreType.DMA,
            pltpu.SemaphoreType.DMA,
            pltpu.SemaphoreType.DMA,
            pltpu.SemaphoreType.DMA,
            pltpu.SemaphoreType.DMA,
        ],
    )(x0_cat, edge_index)


def kernel(user_emb, item_emb, edge_index):
    pad = jnp.zeros((NNP - NN, D), jnp.float32)
    x0_cat = jnp.concatenate([user_emb, item_emb, pad], axis=0)
    out = _lightgcn_sc(x0_cat, edge_index)
    return out[:N_USERS], out[N_USERS:NN]
